# baseline (speedup 1.0000x reference)
import functools

import jax
import jax.numpy as jnp
from jax import lax
from jax.experimental import pallas as pl
from jax.experimental.pallas import tpu as pltpu
from jax.experimental.pallas import tpu_sc as plsc

NUM_ROWS = 1000000
DIM = 64

IDX_PER_DMA = 128
GATHERS_PER_BLK = 8
BLK = IDX_PER_DMA * GATHERS_PER_BLK


def _make_kernel(batch, num_workers):
    assert batch % (num_workers * BLK) == 0
    b_per_w = batch // num_workers
    n_blk = b_per_w // BLK
    mesh = plsc.VectorSubcoreMesh(core_axis_name="c", subcore_axis_name="s")
    nc = plsc.get_sparse_core_info().num_cores

    @functools.partial(
        pl.kernel,
        mesh=mesh,
        out_type=jax.ShapeDtypeStruct((batch, DIM), jnp.float32),
        scratch_types=[
            pltpu.VMEM((GATHERS_PER_BLK, IDX_PER_DMA), jnp.int32),
            pltpu.VMEM((BLK, DIM), jnp.float32),
            pltpu.SemaphoreType.DMA,
        ],
        compiler_params=pltpu.CompilerParams(use_tc_tiling_on_sc=False),
    )
    def emb_kernel(idx_hbm, table_hbm, out_hbm, idx_v, rows_v, sem):
        wid = lax.axis_index("s") * nc + lax.axis_index("c")
        row_base = wid * b_per_w
        chunk_base = row_base // IDX_PER_DMA

        def body(g, _):
            pltpu.sync_copy(
                idx_hbm.at[
                    pl.ds(
                        pl.multiple_of(chunk_base + g * GATHERS_PER_BLK, 8),
                        GATHERS_PER_BLK,
                    )
                ],
                idx_v,
            )
            copies = [
                pltpu.make_async_copy(
                    table_hbm.at[idx_v.at[j]],
                    rows_v.at[pl.ds(j * IDX_PER_DMA, IDX_PER_DMA)],
                    sem,
                )
                for j in range(GATHERS_PER_BLK)
            ]
            for c in copies:
                c.start()
            for c in copies:
                c.wait()
            pltpu.sync_copy(rows_v, out_hbm.at[pl.ds(row_base + g * BLK, BLK)])
            return 0

        lax.fori_loop(0, n_blk, body, 0)

    return emb_kernel


def kernel(indices, weight):
    batch = indices.size
    idx_flat = indices.reshape(batch // IDX_PER_DMA, IDX_PER_DMA).astype(jnp.int32)
    out = _make_kernel(batch, 32)(idx_flat, weight)
    return out.reshape(*indices.shape, DIM)


# speedup vs baseline: 1.0260x; 1.0260x over previous
import functools

import jax
import jax.numpy as jnp
from jax import lax
from jax.experimental import pallas as pl
from jax.experimental.pallas import tpu as pltpu
from jax.experimental.pallas import tpu_sc as plsc

DIM = 64

IDX_PER_DMA = 128
GATHERS_PER_BLK = 5
BLK = IDX_PER_DMA * GATHERS_PER_BLK


def _make_kernel(batch, num_workers):
    assert batch % (num_workers * 2 * BLK) == 0
    b_per_w = batch // num_workers
    n_pair = b_per_w // (2 * BLK)
    mesh = plsc.VectorSubcoreMesh(core_axis_name="c", subcore_axis_name="s")
    nc = plsc.get_sparse_core_info().num_cores

    @functools.partial(
        pl.kernel,
        mesh=mesh,
        out_type=jax.ShapeDtypeStruct((batch, DIM), jnp.float32),
        scratch_types=[
            pltpu.VMEM((2, BLK), jnp.int32),
            pltpu.VMEM((2, BLK, DIM), jnp.float32),
            pltpu.SemaphoreType.DMA,
            pltpu.SemaphoreType.DMA,
            pltpu.SemaphoreType.DMA,
            pltpu.SemaphoreType.DMA,
        ],
        compiler_params=pltpu.CompilerParams(use_tc_tiling_on_sc=False),
    )
    def emb_kernel(idx_hbm, table_hbm, out_hbm, idx_v, rows_v, sg0, sg1, ss0, ss1):
        wid = lax.axis_index("s") * nc + lax.axis_index("c")
        row_base = wid * b_per_w
        sg = (sg0, sg1)
        ss = (ss0, ss1)

        def gathers(g, b, sem):
            off = pl.multiple_of(row_base + g * BLK, 128)
            pltpu.sync_copy(idx_hbm.at[pl.ds(off, BLK)], idx_v.at[b])
            cps = [
                pltpu.make_async_copy(
                    table_hbm.at[idx_v.at[b, pl.ds(j * IDX_PER_DMA, IDX_PER_DMA)]],
                    rows_v.at[b, pl.ds(j * IDX_PER_DMA, IDX_PER_DMA)],
                    sem,
                )
                for j in range(GATHERS_PER_BLK)
            ]
            for c in cps:
                c.start()
            return cps

        def store(g, b, sem):
            off = pl.multiple_of(row_base + g * BLK, 128)
            cp = pltpu.make_async_copy(rows_v.at[b], out_hbm.at[pl.ds(off, BLK)], sem)
            cp.start()
            return cp

        pre0 = gathers(0, 0, sg[0])
        pre1 = gathers(1, 1, sg[1])

        def body(t, _):
            g0 = 2 * t
            for c in pre0:
                c.wait()
            st0 = store(g0, 0, ss[0])
            for c in pre1:
                c.wait()
            st1 = store(g0 + 1, 1, ss[1])
            st0.wait()

            @pl.when(t < n_pair - 1)
            def _():
                gathers(g0 + 2, 0, sg[0])

            st1.wait()

            @pl.when(t < n_pair - 1)
            def _():
                gathers(g0 + 3, 1, sg[1])

            return 0

        lax.fori_loop(0, n_pair, body, 0)

    return emb_kernel


def kernel(indices, weight):
    batch = indices.size
    idx_flat = indices.reshape(batch).astype(jnp.int32)
    out = _make_kernel(batch, 32)(idx_flat, weight)
    return out.reshape(*indices.shape, DIM)
